# baseline (speedup 1.0000x reference)
import functools

import jax
import jax.numpy as jnp
from jax import lax
from jax.experimental import pallas as pl
from jax.experimental.pallas import tpu as pltpu
from jax.experimental.pallas import tpu_sc as plsc

N_NODES = 10000
MAX_DEG = 32
ADJW = 128
D = 128
B = 1024
S1 = 10
S2 = 25
NC, NS = 2, 16
NW = NC * NS
BPW = B // NW
NL = 16
DC = D // NL


def _sum25(buf):
    accs = tuple(buf[0, pl.ds(c * NL, NL)] for c in range(DC))

    def chunk(r, accs):
        return tuple(a + buf[r, pl.ds(c * NL, NL)]
                     for c, a in enumerate(accs))

    return lax.fori_loop(1, S2, chunk, accs)


GRP = 4
NBUF = 7
LOOK = NBUF - 1


SPCH = 640


def _sc_gather_kernel(features, adj, batch1, fb_out, fs1_out, a1_out,
                      bidx_v, adjb_v, adj1_a, adj1_b, fs1_v,
                      f25_bufs, a1_v, spm,
                      sem_b, sem_fb, sem_a0, sem_a1, semf_arr,
                      sem_f10, sem_w0, sem_w1):
    sid = lax.axis_index("s")
    wid = sid * NC + lax.axis_index("c")
    base = wid * BPW

    pltpu.sync_copy(batch1.at[pl.ds(base, BPW)], bidx_v)
    cp_adj = pltpu.async_copy(adj.at[bidx_v], adjb_v, sem_b)

    off = sid * SPCH

    @pl.when(sid < NS - 1)
    def _():
        pltpu.sync_copy(features.at[pl.ds(off, SPCH)], spm.at[pl.ds(off, SPCH)])

    @pl.when(sid == NS - 1)
    def _():
        pltpu.sync_copy(features.at[pl.ds(off, N_NODES - (NS - 1) * SPCH)],
                        spm.at[pl.ds(off, N_NODES - (NS - 1) * SPCH)])

    plsc.subcore_barrier()

    cp_adj.wait()

    adj1 = (adj1_a, adj1_b)
    sema = (sem_a0, sem_a1)
    f25 = [f25_bufs.at[k] for k in range(NBUF)]
    semf = [semf_arr.at[k] for k in range(NBUF)]
    inv25 = jnp.float32(1.0 / S2)
    T = GRP * S1

    def g_body(g, carry):
        b0 = g * GRP

        pend = [None] * GRP
        pend[0] = pltpu.async_copy(
            adj.at[adjb_v.at[b0, pl.ds(0, S1)]], adj1[0], sema[0])
        pend[1] = pltpu.async_copy(
            adj.at[adjb_v.at[b0 + 1, pl.ds(0, S1)]], adj1[1], sema[1])

        @pl.when(g > 0)
        def _():
            pltpu.make_async_copy(
                fs1_v, fs1_out.at[pl.ds(base * S1, T)], sem_w0).wait()
            pltpu.make_async_copy(
                a1_v, a1_out.at[pl.ds(base * S1, T)], sem_w1).wait()

        cpf = []
        for bb in range(GRP):
            idxb = adjb_v.at[b0 + bb, pl.ds(0, S1)]
            cpf.append(pltpu.async_copy(
                spm.at[idxb], fs1_v.at[pl.ds(bb * S1, S1)], sem_f10))

        pend[0].wait()

        cps = [None] * NBUF

        def issue_f25(t):
            bb, j = divmod(t, S1)
            cps[t % NBUF] = pltpu.async_copy(
                spm.at[adj1[bb % 2].at[j, pl.ds(0, S2)]],
                f25[t % NBUF], semf[t % NBUF])

        for t in range(LOOK):
            issue_f25(t)
        for t in range(T):
            bb, j = divmod(t, S1)
            ti = t + LOOK
            if ti < T:
                nb, nj = divmod(ti, S1)
                if nj == 0:
                    pend[nb].wait()
                issue_f25(ti)
            cps[t % NBUF].wait()
            if j == S1 - 1 and bb + 2 < GRP:
                pend[bb + 2] = pltpu.async_copy(
                    adj.at[adjb_v.at[b0 + bb + 2, pl.ds(0, S1)]],
                    adj1[(bb + 2) % 2], sema[(bb + 2) % 2])
            accs = _sum25(f25[t % NBUF])
            for c in range(DC):
                a1_v[bb * S1 + j, pl.ds(c * NL, NL)] = accs[c] * inv25
        for cp in cpf:
            cp.wait()
        row0 = (base + b0) * S1
        pltpu.async_copy(fs1_v, fs1_out.at[pl.ds(row0, T)], sem_w0)
        pltpu.async_copy(a1_v, a1_out.at[pl.ds(row0, T)], sem_w1)
        return carry

    lax.fori_loop(0, BPW // GRP, g_body, 0)

    pltpu.make_async_copy(fs1_v, fs1_out.at[pl.ds(base * S1, T)], sem_w0).wait()
    pltpu.make_async_copy(a1_v, a1_out.at[pl.ds(base * S1, T)], sem_w1).wait()
    pltpu.async_copy(spm.at[bidx_v], fs1_v.at[pl.ds(0, BPW)], sem_fb).wait()
    pltpu.sync_copy(fs1_v.at[pl.ds(0, BPW)], fb_out.at[pl.ds(base, BPW)])


def _sc_gather(features, adj, batch1):
    mesh = plsc.VectorSubcoreMesh(core_axis_name="c", subcore_axis_name="s")
    f32 = jnp.float32
    kern = functools.partial(
        pl.kernel,
        mesh=mesh,
        out_type=[
            jax.ShapeDtypeStruct((B, D), f32),
            jax.ShapeDtypeStruct((B * S1, D), f32),
            jax.ShapeDtypeStruct((B * S1, D), f32),
        ],
        scratch_types=[
            pltpu.VMEM((BPW,), jnp.int32),
            pltpu.VMEM((BPW, ADJW), jnp.int32),
            pltpu.VMEM((S1, ADJW), jnp.int32),
            pltpu.VMEM((S1, ADJW), jnp.int32),
            pltpu.VMEM((GRP * S1, D), f32),
            pltpu.VMEM((NBUF, S2, D), f32),
            pltpu.VMEM((GRP * S1, D), f32),
            pltpu.VMEM_SHARED((N_NODES, D), f32),
            pltpu.SemaphoreType.DMA,
            pltpu.SemaphoreType.DMA,
            pltpu.SemaphoreType.DMA,
            pltpu.SemaphoreType.DMA,
            pltpu.SemaphoreType.DMA((NBUF,)),
            pltpu.SemaphoreType.DMA,
            pltpu.SemaphoreType.DMA,
            pltpu.SemaphoreType.DMA,
        ],
    )(_sc_gather_kernel)
    return kern(features, adj, batch1)


BLK = 8
RPB = B // BLK
SPB = RPB * S1


def _tc_dense_kernel(fs1_ref, a1_ref, fb_ref, ws0_ref, wn0_ref, ws1_ref,
                     wn1_ref, out_ref):
    f32 = jnp.float32
    fs1 = fs1_ref[...]
    a1 = a1_ref[...]
    ws0 = ws0_ref[...]
    wn0 = wn0_ref[...]

    h1 = jnp.concatenate(
        [jnp.dot(fs1, ws0, preferred_element_type=f32),
         jnp.dot(a1, wn0, preferred_element_type=f32)], axis=1)
    h1 = jnp.maximum(h1, 0.0)

    row = lax.broadcasted_iota(jnp.int32, (RPB, SPB), 0)
    col = lax.broadcasted_iota(jnp.int32, (RPB, SPB), 1)
    G = jnp.where(col // S1 == row, f32(1.0 / S1), f32(0.0))

    a0 = jnp.dot(G, fs1, preferred_element_type=f32)
    m = jnp.dot(G, h1, preferred_element_type=f32)
    h0 = jnp.concatenate(
        [jnp.dot(fb_ref[...], ws0, preferred_element_type=f32),
         jnp.dot(a0, wn0, preferred_element_type=f32)], axis=1)
    h0 = jnp.maximum(h0, 0.0)

    o = jnp.concatenate(
        [jnp.dot(h0, ws1_ref[...], preferred_element_type=f32),
         jnp.dot(m, wn1_ref[...], preferred_element_type=f32)], axis=1)
    norm = jnp.sqrt(jnp.sum(o * o, axis=1, keepdims=True))
    out_ref[...] = o / jnp.maximum(norm, 1e-12)


def _tc_dense(fb, fs1, a1, ws0, wn0, ws1, wn1):
    f32 = jnp.float32
    return pl.pallas_call(
        _tc_dense_kernel,
        grid=(BLK,),
        in_specs=[
            pl.BlockSpec((SPB, D), lambda i: (i, 0)),
            pl.BlockSpec((SPB, D), lambda i: (i, 0)),
            pl.BlockSpec((RPB, D), lambda i: (i, 0)),
            pl.BlockSpec((D, 128), lambda i: (0, 0)),
            pl.BlockSpec((D, 128), lambda i: (0, 0)),
            pl.BlockSpec((256, 128), lambda i: (0, 0)),
            pl.BlockSpec((256, 128), lambda i: (0, 0)),
        ],
        out_specs=pl.BlockSpec((RPB, 256), lambda i: (i, 0)),
        out_shape=jax.ShapeDtypeStruct((B, 256), f32),
    )(fs1, a1, fb, ws0, wn0, ws1, wn1)


def kernel(features, adj, batch1, W_self_0, W_neigh_0, W_self_1, W_neigh_1):
    adjp = jnp.pad(adj, ((0, 0), (0, ADJW - MAX_DEG)))
    fb, fs1, a1 = _sc_gather(features, adjp, batch1)
    return _tc_dense(fb, fs1, a1, W_self_0, W_neigh_0, W_self_1, W_neigh_1)


# speedup vs baseline: 1.0160x; 1.0160x over previous
import functools

import jax
import jax.numpy as jnp
from jax import lax
from jax.experimental import pallas as pl
from jax.experimental.pallas import tpu as pltpu
from jax.experimental.pallas import tpu_sc as plsc

N_NODES = 10000
MAX_DEG = 32
ADJW = 128
D = 128
B = 1024
S1 = 10
S2 = 25
NC, NS = 2, 16
NW = NC * NS
BPW = B // NW
NL = 16
DC = D // NL


def _sum25(buf):
    accs = tuple(buf[0, pl.ds(c * NL, NL)] for c in range(DC))

    def chunk(r, accs):
        return tuple(a + buf[r, pl.ds(c * NL, NL)]
                     for c, a in enumerate(accs))

    return lax.fori_loop(1, S2, chunk, accs)


GRP = 4
NBUF = 7
LOOK = NBUF - 1


SPCH = 640


def _sc_gather_kernel(features, adj, batch1, fb_out, fs1_out, a1_out,
                      bidx_v, adjb_v, adj1_a, adj1_b, fs1_v,
                      f25_bufs, a1_v, spm,
                      sem_b, sem_fb, sem_a0, sem_a1, semf_arr,
                      sem_f10, sem_w0, sem_w1):
    sid = lax.axis_index("s")
    wid = sid * NC + lax.axis_index("c")
    base = wid * BPW

    pltpu.sync_copy(batch1.at[pl.ds(base, BPW)], bidx_v)
    cp_adj = pltpu.async_copy(adj.at[bidx_v], adjb_v, sem_b)

    off = sid * SPCH

    @pl.when(sid < NS - 1)
    def _():
        pltpu.sync_copy(features.at[pl.ds(off, SPCH)], spm.at[pl.ds(off, SPCH)])

    @pl.when(sid == NS - 1)
    def _():
        pltpu.sync_copy(features.at[pl.ds(off, N_NODES - (NS - 1) * SPCH)],
                        spm.at[pl.ds(off, N_NODES - (NS - 1) * SPCH)])

    plsc.subcore_barrier()

    cp_adj.wait()

    adj1 = (adj1_a, adj1_b)
    sema = (sem_a0, sem_a1)
    f25 = [f25_bufs.at[k] for k in range(NBUF)]
    semf = [semf_arr.at[k] for k in range(NBUF)]
    inv25 = jnp.float32(1.0 / S2)
    T = GRP * S1

    def g_body(g, carry):
        b0 = g * GRP

        pend = [None] * GRP
        pend[0] = pltpu.async_copy(
            adj.at[adjb_v.at[b0, pl.ds(0, S1)]], adj1[0], sema[0])
        pend[1] = pltpu.async_copy(
            adj.at[adjb_v.at[b0 + 1, pl.ds(0, S1)]], adj1[1], sema[1])

        @pl.when(g > 0)
        def _():
            pltpu.make_async_copy(
                fs1_v, fs1_out.at[pl.ds(base * S1, T)], sem_w0).wait()
            pltpu.make_async_copy(
                a1_v, a1_out.at[pl.ds(base * S1, T)], sem_w1).wait()

        cpf = []
        for bb in range(GRP):
            idxb = adjb_v.at[b0 + bb, pl.ds(0, S1)]
            cpf.append(pltpu.async_copy(
                spm.at[idxb], fs1_v.at[pl.ds(bb * S1, S1)], sem_f10))

        pend[0].wait()

        cps = [None] * NBUF

        def issue_f25(t):
            bb, j = divmod(t, S1)
            cps[t % NBUF] = pltpu.async_copy(
                spm.at[adj1[bb % 2].at[j, pl.ds(0, S2)]],
                f25[t % NBUF], semf[t % NBUF])

        for t in range(LOOK):
            issue_f25(t)
        for t in range(T):
            bb, j = divmod(t, S1)
            ti = t + LOOK
            if ti < T:
                nb, nj = divmod(ti, S1)
                if nj == 0:
                    pend[nb].wait()
                issue_f25(ti)
            cps[t % NBUF].wait()
            if j == S1 - 1 and bb + 2 < GRP:
                pend[bb + 2] = pltpu.async_copy(
                    adj.at[adjb_v.at[b0 + bb + 2, pl.ds(0, S1)]],
                    adj1[(bb + 2) % 2], sema[(bb + 2) % 2])
            accs = _sum25(f25[t % NBUF])
            for c in range(DC):
                a1_v[bb * S1 + j, pl.ds(c * NL, NL)] = accs[c] * inv25
        for cp in cpf:
            cp.wait()
        row0 = (base + b0) * S1
        pltpu.async_copy(fs1_v, fs1_out.at[pl.ds(row0, T)], sem_w0)
        pltpu.async_copy(a1_v, a1_out.at[pl.ds(row0, T)], sem_w1)
        return carry

    lax.fori_loop(0, BPW // GRP, g_body, 0)

    pltpu.make_async_copy(fs1_v, fs1_out.at[pl.ds(base * S1, T)], sem_w0).wait()
    pltpu.make_async_copy(a1_v, a1_out.at[pl.ds(base * S1, T)], sem_w1).wait()
    pltpu.async_copy(spm.at[bidx_v], fs1_v.at[pl.ds(0, BPW)], sem_fb).wait()
    pltpu.sync_copy(fs1_v.at[pl.ds(0, BPW)], fb_out.at[pl.ds(base, BPW)])


def _sc_gather(features, adj, batch1):
    mesh = plsc.VectorSubcoreMesh(core_axis_name="c", subcore_axis_name="s")
    f32 = jnp.float32
    kern = functools.partial(
        pl.kernel,
        mesh=mesh,
        out_type=[
            jax.ShapeDtypeStruct((B, D), f32),
            jax.ShapeDtypeStruct((B * S1, D), f32),
            jax.ShapeDtypeStruct((B * S1, D), f32),
        ],
        scratch_types=[
            pltpu.VMEM((BPW,), jnp.int32),
            pltpu.VMEM((BPW, ADJW), jnp.int32),
            pltpu.VMEM((S1, ADJW), jnp.int32),
            pltpu.VMEM((S1, ADJW), jnp.int32),
            pltpu.VMEM((GRP * S1, D), f32),
            pltpu.VMEM((NBUF, S2, D), f32),
            pltpu.VMEM((GRP * S1, D), f32),
            pltpu.VMEM_SHARED((N_NODES, D), f32),
            pltpu.SemaphoreType.DMA,
            pltpu.SemaphoreType.DMA,
            pltpu.SemaphoreType.DMA,
            pltpu.SemaphoreType.DMA,
            pltpu.SemaphoreType.DMA((NBUF,)),
            pltpu.SemaphoreType.DMA,
            pltpu.SemaphoreType.DMA,
            pltpu.SemaphoreType.DMA,
        ],
    )(_sc_gather_kernel)
    return kern(features, adj, batch1)


BLK = 4
RPB = B // BLK
SPB = RPB * S1


def _tc_dense_kernel(fs1_ref, a1_ref, fb_ref, ws0_ref, wn0_ref, ws1_ref,
                     wn1_ref, out_ref):
    f32 = jnp.float32
    fs1 = fs1_ref[...]
    a1 = a1_ref[...]
    ws0 = ws0_ref[...]
    wn0 = wn0_ref[...]

    h1 = jnp.concatenate(
        [jnp.dot(fs1, ws0, preferred_element_type=f32),
         jnp.dot(a1, wn0, preferred_element_type=f32)], axis=1)
    h1 = jnp.maximum(h1, 0.0)

    row = lax.broadcasted_iota(jnp.int32, (RPB, SPB), 0)
    col = lax.broadcasted_iota(jnp.int32, (RPB, SPB), 1)
    G = jnp.where(col // S1 == row, f32(1.0 / S1), f32(0.0))

    a0 = jnp.dot(G, fs1, preferred_element_type=f32)
    m = jnp.dot(G, h1, preferred_element_type=f32)
    h0 = jnp.concatenate(
        [jnp.dot(fb_ref[...], ws0, preferred_element_type=f32),
         jnp.dot(a0, wn0, preferred_element_type=f32)], axis=1)
    h0 = jnp.maximum(h0, 0.0)

    o = jnp.concatenate(
        [jnp.dot(h0, ws1_ref[...], preferred_element_type=f32),
         jnp.dot(m, wn1_ref[...], preferred_element_type=f32)], axis=1)
    norm = jnp.sqrt(jnp.sum(o * o, axis=1, keepdims=True))
    out_ref[...] = o / jnp.maximum(norm, 1e-12)


def _tc_dense(fb, fs1, a1, ws0, wn0, ws1, wn1):
    f32 = jnp.float32
    return pl.pallas_call(
        _tc_dense_kernel,
        grid=(BLK,),
        in_specs=[
            pl.BlockSpec((SPB, D), lambda i: (i, 0)),
            pl.BlockSpec((SPB, D), lambda i: (i, 0)),
            pl.BlockSpec((RPB, D), lambda i: (i, 0)),
            pl.BlockSpec((D, 128), lambda i: (0, 0)),
            pl.BlockSpec((D, 128), lambda i: (0, 0)),
            pl.BlockSpec((256, 128), lambda i: (0, 0)),
            pl.BlockSpec((256, 128), lambda i: (0, 0)),
        ],
        out_specs=pl.BlockSpec((RPB, 256), lambda i: (i, 0)),
        out_shape=jax.ShapeDtypeStruct((B, 256), f32),
    )(fs1, a1, fb, ws0, wn0, ws1, wn1)


def kernel(features, adj, batch1, W_self_0, W_neigh_0, W_self_1, W_neigh_1):
    adjp = jnp.pad(adj, ((0, 0), (0, ADJW - MAX_DEG)))
    fb, fs1, a1 = _sc_gather(features, adjp, batch1)
    return _tc_dense(fb, fs1, a1, W_self_0, W_neigh_0, W_self_1, W_neigh_1)
